# Initial kernel scaffold; baseline (speedup 1.0000x reference)
import jax
import jax.numpy as jnp
from jax.experimental import pallas as pl


def kernel(x, edge_index, edge_weight, pred_edge_index, W1s, W1a, b1, We1, be1, We2, be2, W2s, W2a, b2, Wp1, bp1, Wp2, bp2):
    raise NotImplementedError("write your pallas kernel here")


# speedup vs baseline: 1.4583x; 1.4583x over previous
import jax
import jax.numpy as jnp
from jax import lax
from jax.experimental import pallas as pl
from jax.experimental.pallas import tpu as pltpu
from jax.experimental.pallas import tpu_sc as plsc

NC = 2
NS = 16
NW = NC * NS
L = 16
CH = 80
THR = 1e-3
F32 = jnp.float32

_SC_PARAMS = pltpu.CompilerParams(needs_layout_passes=False)


def _worker_id():
    cid = lax.axis_index("c")
    sid = lax.axis_index("s")
    return cid, sid, sid * NC + cid


def _sc_mesh():
    return plsc.VectorSubcoreMesh(core_axis_name="c", subcore_axis_name="s")


def _scale_rows(rows, s_v, idx_d=None, deg_t=None, mask0=None):
    nj = rows.shape[1] // L

    def body(e, _):
        e16 = jnp.full((L,), e, jnp.int32)
        sb = plsc.load_gather(s_v, [e16])
        for j in range(nj):
            sl = pl.ds(j * L, L)
            rows[e, sl] = rows[e, sl] * sb
        if deg_t is not None:
            db = plsc.load_gather(idx_d, [e16])
            plsc.addupdate_scatter(deg_t, [db], jnp.abs(sb), mask=mask0)
        return 0

    lax.fori_loop(0, rows.shape[0], body, 0)


def _zero_vmem_1d(buf):
    zv = jnp.zeros((L,), F32)

    def body(i, _):
        buf[pl.ds(i * L, L)] = zv
        return 0

    lax.fori_loop(0, buf.shape[0] // L, body, 0)


def _zero_vmem_2d(buf):
    zv = jnp.zeros((L,), F32)
    nr, wdt = buf.shape

    def body(i, _):
        for j in range(wdt // L):
            buf[i, pl.ds(j * L, L)] = zv
        return 0

    lax.fori_loop(0, nr, body, 0)


def _fill_ramp(idxbuf, base):
    iota = lax.iota(jnp.int32, L)

    def b(i, _):
        idxbuf[pl.ds(i * L, L)] = iota + (base + i * L)
        return 0

    lax.fori_loop(0, idxbuf.shape[0] // L, b, 0)


def _spmem_init_zero(idxbuf, sid, zbufs_shs):
    nr = idxbuf.shape[0]
    rps = zbufs_shs[0][1].shape[0] // NS

    def blk(k, _):
        _fill_ramp(idxbuf, sid * rps + k * nr)
        for zbuf, sh in zbufs_shs:
            pltpu.sync_copy(zbuf, sh.at[idxbuf])
        return 0

    lax.fori_loop(0, rps // nr, blk, 0)


def _spmem_readout(idxbuf, cid, sid, bufs_shs_outs):
    nr = idxbuf.shape[0]
    rps = bufs_shs_outs[0][1].shape[0] // NS

    def blk(k, _):
        base = sid * rps + k * nr
        _fill_ramp(idxbuf, base)
        for buf, sh, out in bufs_shs_outs:
            pltpu.sync_copy(sh.at[idxbuf], buf)
            pltpu.sync_copy(buf, out.at[cid, pl.ds(base, nr)])
        return 0

    lax.fori_loop(0, rps // nr, blk, 0)


def _sc_agg_body(table, src, dst, w,
                 agg_out, deg_out,
                 idx_s, idx_d, w_v, rows, deg_t, sh_agg):
    cid, sid, wid = _worker_id()
    ch = idx_s.shape[0]
    epw = src.shape[0] // NW
    nch = epw // ch

    _zero_vmem_2d(rows)
    _zero_vmem_1d(deg_t)
    _spmem_init_zero(idx_s, sid, [(rows, sh_agg)])
    plsc.subcore_barrier()

    iota = lax.iota(jnp.int32, L)
    mask0 = iota == 0
    base0 = wid * epw

    def chunk(c, _):
        b = base0 + c * ch
        pltpu.sync_copy(src.at[pl.ds(b, ch)], idx_s)
        pltpu.sync_copy(dst.at[pl.ds(b, ch)], idx_d)
        pltpu.sync_copy(w.at[pl.ds(b, ch)], w_v)
        pltpu.sync_copy(table.at[idx_s], rows)
        _scale_rows(rows, w_v, idx_d, deg_t, mask0)
        pltpu.sync_copy(rows, sh_agg.at[idx_d], add=True)
        return 0

    lax.fori_loop(0, nch, chunk, 0)
    pltpu.sync_copy(deg_t, deg_out.at[wid])
    plsc.subcore_barrier()
    _spmem_readout(idx_s, cid, sid, [(rows, sh_agg, agg_out)])


def _sc_rw_body(a_t, b_t, src, dst, w, we2, be2,
                rw_out, l1_out,
                idx_s, idx_d, w_v, rw_v, rows_a, rows_b,
                we2_v, be2_v, l1buf):
    _, _, wid = _worker_id()
    h = a_t.shape[1]
    ch = idx_s.shape[0]
    epw = src.shape[0] // NW
    nch = epw // ch

    pltpu.sync_copy(we2, we2_v)
    pltpu.sync_copy(be2, be2_v)

    iota = lax.iota(jnp.int32, L)
    zv = jnp.zeros((L,), F32)
    base0 = wid * epw

    def chunk(c, l1):
        b = base0 + c * ch
        pltpu.sync_copy(src.at[pl.ds(b, ch)], idx_s)
        pltpu.sync_copy(dst.at[pl.ds(b, ch)], idx_d)
        pltpu.sync_copy(w.at[pl.ds(b, ch)], w_v)
        pltpu.sync_copy(a_t.at[idx_s], rows_a)
        pltpu.sync_copy(b_t.at[idx_d], rows_b)

        def group(g, l1g):
            w16 = w_v[pl.ds(g * L, L)]
            eids = iota + g * L

            def feat(f, acc):
                f16 = jnp.full((L,), f, jnp.int32)
                av = plsc.load_gather(rows_a, [eids, f16])
                bv = plsc.load_gather(rows_b, [eids, f16])
                t = jnp.maximum(av + bv, 0.0)
                return acc + t * we2_v[f, pl.ds(0, L)]

            logit = lax.fori_loop(0, h, feat, zv) + be2_v[pl.ds(0, L)]
            p = 1.0 / (1.0 + jnp.exp(-logit))
            rw = w16 * p
            rw = jnp.where(jnp.abs(rw) > THR, rw, 0.0)
            rw_v[pl.ds(g * L, L)] = rw
            return l1g + p

        l1 = lax.fori_loop(0, ch // L, group, l1)
        pltpu.sync_copy(rw_v, rw_out.at[pl.ds(b, ch)])
        return l1

    l1 = lax.fori_loop(0, nch, chunk, zv)
    l1buf[pl.ds(0, L)] = l1
    pltpu.sync_copy(l1buf, l1_out.at[wid])


def _sc_pred_body(u_t, v_t, ps, pd, w0, w1, bp,
                  o0, o1,
                  idx_s, idx_d, rows_u, rows_v, ob0, ob1, w0_v, w1_v, bp_v):
    _, _, wid = _worker_id()
    ch = idx_s.shape[0]
    d2 = u_t.shape[1]
    epw = ps.shape[0] // NW
    nch = epw // ch

    pltpu.sync_copy(w0, w0_v)
    pltpu.sync_copy(w1, w1_v)
    pltpu.sync_copy(bp, bp_v)

    iota = lax.iota(jnp.int32, L)
    zv = jnp.zeros((L,), F32)
    base0 = wid * epw

    def chunk(c, _):
        b = base0 + c * ch
        pltpu.sync_copy(ps.at[pl.ds(b, ch)], idx_s)
        pltpu.sync_copy(pd.at[pl.ds(b, ch)], idx_d)
        pltpu.sync_copy(u_t.at[idx_s], rows_u)
        pltpu.sync_copy(v_t.at[idx_d], rows_v)

        def group(g, _):
            eids = iota + g * L

            def feat(f, accs):
                a0, a1 = accs
                f16 = jnp.full((L,), f, jnp.int32)
                uv = plsc.load_gather(rows_u, [eids, f16])
                vv = plsc.load_gather(rows_v, [eids, f16])
                t = jnp.maximum(uv + vv, 0.0)
                return (a0 + t * w0_v[f, pl.ds(0, L)],
                        a1 + t * w1_v[f, pl.ds(0, L)])

            a0, a1 = lax.fori_loop(0, d2, feat, (zv, zv))
            ob0[pl.ds(g * L, L)] = a0 + bp_v[0, pl.ds(0, L)]
            ob1[pl.ds(g * L, L)] = a1 + bp_v[1, pl.ds(0, L)]
            return 0

        lax.fori_loop(0, ch // L, group, 0)
        pltpu.sync_copy(ob0, o0.at[pl.ds(b, ch)])
        pltpu.sync_copy(ob1, o1.at[pl.ds(b, ch)])
        return 0

    lax.fori_loop(0, nch, chunk, 0)


def _tc1_body(x, aggp, degp, w1s, w1a, b1, we1a, we1b, be1,
              h1_o, a_o, b_o):
    agg = aggp[0] + aggp[1]
    deg = jnp.sum(degp[...], axis=0)
    aggn = agg / jnp.maximum(deg, 1.0)[:, None]
    h1 = jnp.tanh(jnp.dot(x[...], w1s[...], preferred_element_type=F32)
                  + jnp.dot(aggn, w1a[...], preferred_element_type=F32)
                  + b1[...])
    h1_o[...] = h1
    a_o[...] = jnp.dot(h1, we1a[...], preferred_element_type=F32)
    b_o[...] = jnp.dot(h1, we1b[...], preferred_element_type=F32) + be1[...]


def _tc2_body(h1, aggp, degp, w2s, w2a, b2, wp1a, wp1b, wp1c, wp1d, bp1,
              u_o, v_o):
    agg = aggp[0] + aggp[1]
    deg = jnp.sum(degp[...], axis=0)
    aggn = agg / jnp.maximum(deg, 1.0)[:, None]
    h1v = h1[...]
    h2 = jnp.tanh(jnp.dot(h1v, w2s[...], preferred_element_type=F32)
                  + jnp.dot(aggn, w2a[...], preferred_element_type=F32)
                  + b2[...])
    u_o[...] = (jnp.dot(h1v, wp1a[...], preferred_element_type=F32)
                + jnp.dot(h2, wp1b[...], preferred_element_type=F32)
                + bp1[...])
    v_o[...] = (jnp.dot(h1v, wp1c[...], preferred_element_type=F32)
                + jnp.dot(h2, wp1d[...], preferred_element_type=F32))


def kernel(x, edge_index, edge_weight, pred_edge_index,
           W1s, W1a, b1, We1, be1, We2, be2,
           W2s, W2a, b2, Wp1, bp1, Wp2, bp2):
    n, d_in = x.shape
    h = W1s.shape[1]
    e = edge_index.shape[1]
    p = pred_edge_index.shape[1]
    d2 = 2 * h
    src, dst = edge_index[0], edge_index[1]
    pps, ppd = pred_edge_index[0], pred_edge_index[1]

    npad = ((n + 511) // 512) * 512
    while (npad // NS) % CH:
        npad += 512

    sc_agg = pl.kernel(
        _sc_agg_body,
        out_type=[jax.ShapeDtypeStruct((NC, npad, h), F32),
                  jax.ShapeDtypeStruct((NW, npad), F32)],
        mesh=_sc_mesh(),
        compiler_params=_SC_PARAMS,
        scratch_types=[
            pltpu.VMEM((CH,), jnp.int32), pltpu.VMEM((CH,), jnp.int32),
            pltpu.VMEM((CH,), F32), pltpu.VMEM((CH, h), F32),
            pltpu.VMEM((npad,), F32),
            pltpu.VMEM_SHARED((npad, h), F32),
        ],
    )
    aggp1, degp1 = sc_agg(x, src, dst, edge_weight)

    rb = 512
    gn = pl.cdiv(n, rb)
    wspec = pl.BlockSpec((d_in, h), lambda i: (0, 0))
    bspec = pl.BlockSpec((1, h), lambda i: (0, 0))
    nspec = pl.BlockSpec((rb, h), lambda i: (i, 0))
    aggspec = pl.BlockSpec((NC, rb, h), lambda i: (0, i, 0))
    degspec = pl.BlockSpec((NW, rb), lambda i: (0, i))

    h1, a_t, b_t = pl.pallas_call(
        _tc1_body,
        grid=(gn,),
        in_specs=[nspec, aggspec, degspec, wspec, wspec, bspec,
                  wspec, wspec, bspec],
        out_specs=[nspec, nspec, nspec],
        out_shape=[jax.ShapeDtypeStruct((n, h), F32)] * 3,
    )(x, aggp1, degp1, W1s, W1a, b1.reshape(1, h),
      We1[:h], We1[h:], be1.reshape(1, h))

    we2b = jnp.broadcast_to(We2.reshape(h, 1), (h, L))
    be2b = jnp.broadcast_to(be2.reshape(1), (L,))
    sc_rw = pl.kernel(
        _sc_rw_body,
        out_type=[jax.ShapeDtypeStruct((e,), F32),
                  jax.ShapeDtypeStruct((NW, L), F32)],
        mesh=_sc_mesh(),
        compiler_params=_SC_PARAMS,
        scratch_types=[
            pltpu.VMEM((CH,), jnp.int32), pltpu.VMEM((CH,), jnp.int32),
            pltpu.VMEM((CH,), F32), pltpu.VMEM((CH,), F32),
            pltpu.VMEM((CH, h), F32), pltpu.VMEM((CH, h), F32),
            pltpu.VMEM((h, L), F32), pltpu.VMEM((L,), F32),
            pltpu.VMEM((L,), F32),
        ],
    )
    rw, l1p = sc_rw(a_t, b_t, src, dst, edge_weight, we2b, be2b)

    sc_agg2 = pl.kernel(
        _sc_agg_body,
        out_type=[jax.ShapeDtypeStruct((NC, npad, h), F32),
                  jax.ShapeDtypeStruct((NW, npad), F32)],
        mesh=_sc_mesh(),
        compiler_params=_SC_PARAMS,
        scratch_types=[
            pltpu.VMEM((CH,), jnp.int32), pltpu.VMEM((CH,), jnp.int32),
            pltpu.VMEM((CH,), F32), pltpu.VMEM((CH, h), F32),
            pltpu.VMEM((npad,), F32),
            pltpu.VMEM_SHARED((npad, h), F32),
        ],
    )
    aggp2, degp2 = sc_agg2(h1, src, dst, rw)

    n2spec = pl.BlockSpec((rb, d2), lambda i: (i, 0))
    w2spec = pl.BlockSpec((h, d2), lambda i: (0, 0))
    b2spec = pl.BlockSpec((1, d2), lambda i: (0, 0))
    u_t, v_t = pl.pallas_call(
        _tc2_body,
        grid=(gn,),
        in_specs=[nspec, aggspec, degspec, wspec, wspec, bspec,
                  w2spec, w2spec, w2spec, w2spec, b2spec],
        out_specs=[n2spec, n2spec],
        out_shape=[jax.ShapeDtypeStruct((n, d2), F32)] * 2,
    )(h1, aggp2, degp2, W2s, W2a, b2.reshape(1, h),
      Wp1[:h], Wp1[h:d2], Wp1[d2:d2 + h], Wp1[d2 + h:], bp1.reshape(1, d2))

    w0b = jnp.broadcast_to(Wp2[:, 0:1], (d2, L))
    w1b = jnp.broadcast_to(Wp2[:, 1:2], (d2, L))
    bpb = jnp.broadcast_to(bp2.reshape(2, 1), (2, L))
    sc_pred = pl.kernel(
        _sc_pred_body,
        out_type=[jax.ShapeDtypeStruct((p,), F32),
                  jax.ShapeDtypeStruct((p,), F32)],
        mesh=_sc_mesh(),
        compiler_params=_SC_PARAMS,
        scratch_types=[
            pltpu.VMEM((CH,), jnp.int32), pltpu.VMEM((CH,), jnp.int32),
            pltpu.VMEM((CH, d2), F32), pltpu.VMEM((CH, d2), F32),
            pltpu.VMEM((CH,), F32), pltpu.VMEM((CH,), F32),
            pltpu.VMEM((d2, L), F32), pltpu.VMEM((d2, L), F32),
            pltpu.VMEM((2, L), F32),
        ],
    )
    o0, o1 = sc_pred(u_t, v_t, pps, ppd, w0b, w1b, bpb)

    logits = jnp.stack([o0, o1], axis=1)
    l1_reg = jnp.sum(l1p) / e
    return logits, l1_reg


# speedup vs baseline: 1.5731x; 1.0787x over previous
import jax
import jax.numpy as jnp
from jax import lax
from jax.experimental import pallas as pl
from jax.experimental.pallas import tpu as pltpu
from jax.experimental.pallas import tpu_sc as plsc

NC = 2
NS = 16
NW = NC * NS
L = 16
CH = 80
THR = 1e-3
F32 = jnp.float32

_SC_PARAMS = pltpu.CompilerParams(needs_layout_passes=False)


def _worker_id():
    cid = lax.axis_index("c")
    sid = lax.axis_index("s")
    return cid, sid, sid * NC + cid


def _sc_mesh():
    return plsc.VectorSubcoreMesh(core_axis_name="c", subcore_axis_name="s")


def _scale_rows(rows, s_v, idx_d=None, deg_t=None, mask0=None):
    nj = rows.shape[1] // L

    def body(e, _):
        e16 = jnp.full((L,), e, jnp.int32)
        sb = plsc.load_gather(s_v, [e16])
        for j in range(nj):
            sl = pl.ds(j * L, L)
            rows[e, sl] = rows[e, sl] * sb
        if deg_t is not None:
            db = plsc.load_gather(idx_d, [e16])
            plsc.addupdate_scatter(deg_t, [db], jnp.abs(sb), mask=mask0)
        return 0

    lax.fori_loop(0, rows.shape[0], body, 0)


def _zero_vmem_1d(buf):
    zv = jnp.zeros((L,), F32)

    def body(i, _):
        buf[pl.ds(i * L, L)] = zv
        return 0

    lax.fori_loop(0, buf.shape[0] // L, body, 0)


def _zero_vmem_2d(buf):
    zv = jnp.zeros((L,), F32)
    nr, wdt = buf.shape

    def body(i, _):
        for j in range(wdt // L):
            buf[i, pl.ds(j * L, L)] = zv
        return 0

    lax.fori_loop(0, nr, body, 0)


def _fill_ramp(idxbuf, base):
    iota = lax.iota(jnp.int32, L)

    def b(i, _):
        idxbuf[pl.ds(i * L, L)] = iota + (base + i * L)
        return 0

    lax.fori_loop(0, idxbuf.shape[0] // L, b, 0)


def _spmem_init_zero(idxbuf, sid, zbufs_shs):
    nr = idxbuf.shape[0]
    rps = zbufs_shs[0][1].shape[0] // NS

    def blk(k, _):
        _fill_ramp(idxbuf, sid * rps + k * nr)
        for zbuf, sh in zbufs_shs:
            pltpu.sync_copy(zbuf, sh.at[idxbuf])
        return 0

    lax.fori_loop(0, rps // nr, blk, 0)


def _spmem_readout(idxbuf, cid, sid, bufs_shs_outs):
    nr = idxbuf.shape[0]
    rps = bufs_shs_outs[0][1].shape[0] // NS

    def blk(k, _):
        base = sid * rps + k * nr
        _fill_ramp(idxbuf, base)
        for buf, sh, out in bufs_shs_outs:
            pltpu.sync_copy(sh.at[idxbuf], buf)
            pltpu.sync_copy(buf, out.at[cid, pl.ds(base, nr)])
        return 0

    lax.fori_loop(0, rps // nr, blk, 0)


def _sc_agg_body(table, src, dst, w,
                 agg_out, deg_out,
                 idx_s, idx_d, w_v, rows, deg_t, sh_agg):
    cid, sid, wid = _worker_id()
    ch = idx_s.shape[0]
    epw = src.shape[0] // NW
    nch = epw // ch

    _zero_vmem_2d(rows)
    _zero_vmem_1d(deg_t)
    _spmem_init_zero(idx_s, sid, [(rows, sh_agg)])
    plsc.subcore_barrier()

    iota = lax.iota(jnp.int32, L)
    mask0 = iota == 0
    base0 = wid * epw

    def chunk(c, _):
        b = base0 + c * ch
        pltpu.sync_copy(src.at[pl.ds(b, ch)], idx_s)
        pltpu.sync_copy(dst.at[pl.ds(b, ch)], idx_d)
        pltpu.sync_copy(w.at[pl.ds(b, ch)], w_v)
        pltpu.sync_copy(table.at[idx_s], rows)
        _scale_rows(rows, w_v, idx_d, deg_t, mask0)
        pltpu.sync_copy(rows, sh_agg.at[idx_d], add=True)
        return 0

    lax.fori_loop(0, nch, chunk, 0)
    pltpu.sync_copy(deg_t, deg_out.at[wid])
    plsc.subcore_barrier()
    _spmem_readout(idx_s, cid, sid, [(rows, sh_agg, agg_out)])


def _rw_compute(i_s, i_d, w_vp, ra, rb, c, l1, base0,
                we2_v, be2_v, rw_v, rw_out, iota, zv):
    ch = i_s.shape[0]
    h = we2_v.shape[0]

    def group(g, l1g):
        w16 = w_vp[pl.ds(g * L, L)]
        eids = iota + g * L

        def feat(fq, acc):
            for u in range(4):
                f = fq * 4 + u
                f16 = jnp.full((L,), f, jnp.int32)
                av = plsc.load_gather(ra, [eids, f16])
                bv = plsc.load_gather(rb, [eids, f16])
                t = jnp.maximum(av + bv, 0.0)
                acc = acc + t * we2_v[f, pl.ds(0, L)]
            return acc

        logit = lax.fori_loop(0, h // 4, feat, zv) + be2_v[pl.ds(0, L)]
        pv = 1.0 / (1.0 + jnp.exp(-logit))
        rw = w16 * pv
        rw = jnp.where(jnp.abs(rw) > THR, rw, 0.0)
        rw_v[pl.ds(g * L, L)] = rw
        return l1g + pv

    l1 = lax.fori_loop(0, ch // L, group, l1)
    pltpu.sync_copy(rw_v, rw_out.at[pl.ds(base0 + c * ch, ch)])
    return l1


def _sc_rw_body(a_t, b_t, src, dst, w, we2, be2,
                rw_out, l1_out,
                i_s0, i_d0, w_v0, i_s1, i_d1, w_v1,
                ra0, rb0, ra1, rb1, rw_v, we2_v, be2_v, l1buf,
                sem_i, sem_g0, sem_g1):
    _, _, wid = _worker_id()
    ch = i_s0.shape[0]
    epw = src.shape[0] // NW
    nch = epw // ch

    pltpu.sync_copy(we2, we2_v)
    pltpu.sync_copy(be2, be2_v)

    iota = lax.iota(jnp.int32, L)
    zv = jnp.zeros((L,), F32)
    base0 = wid * epw

    def pair(k, l1):
        b0 = base0 + (2 * k) * ch
        b1 = b0 + ch
        di = [pltpu.async_copy(src.at[pl.ds(b0, ch)], i_s0, sem_i),
              pltpu.async_copy(dst.at[pl.ds(b0, ch)], i_d0, sem_i),
              pltpu.async_copy(w.at[pl.ds(b0, ch)], w_v0, sem_i),
              pltpu.async_copy(src.at[pl.ds(b1, ch)], i_s1, sem_i),
              pltpu.async_copy(dst.at[pl.ds(b1, ch)], i_d1, sem_i),
              pltpu.async_copy(w.at[pl.ds(b1, ch)], w_v1, sem_i)]
        for d in di:
            d.wait()
        g0 = [pltpu.async_copy(a_t.at[i_s0], ra0, sem_g0),
              pltpu.async_copy(b_t.at[i_d0], rb0, sem_g0)]
        g1 = [pltpu.async_copy(a_t.at[i_s1], ra1, sem_g1),
              pltpu.async_copy(b_t.at[i_d1], rb1, sem_g1)]
        for d in g0:
            d.wait()
        l1 = _rw_compute(i_s0, i_d0, w_v0, ra0, rb0, 2 * k, l1, base0,
                         we2_v, be2_v, rw_v, rw_out, iota, zv)
        for d in g1:
            d.wait()
        l1 = _rw_compute(i_s1, i_d1, w_v1, ra1, rb1, 2 * k + 1, l1, base0,
                         we2_v, be2_v, rw_v, rw_out, iota, zv)
        return l1

    l1 = lax.fori_loop(0, nch // 2, pair, zv)
    if nch % 2:
        c = nch - 1
        b = base0 + c * ch
        pltpu.sync_copy(src.at[pl.ds(b, ch)], i_s0)
        pltpu.sync_copy(dst.at[pl.ds(b, ch)], i_d0)
        pltpu.sync_copy(w.at[pl.ds(b, ch)], w_v0)
        pltpu.sync_copy(a_t.at[i_s0], ra0)
        pltpu.sync_copy(b_t.at[i_d0], rb0)
        l1 = _rw_compute(i_s0, i_d0, w_v0, ra0, rb0, c, l1, base0,
                         we2_v, be2_v, rw_v, rw_out, iota, zv)

    l1buf[pl.ds(0, L)] = l1
    pltpu.sync_copy(l1buf, l1_out.at[wid])


def _sc_pred_body(u_t, v_t, ps, pd, w0, w1, bp,
                  o0, o1,
                  idx_s, idx_d, rows_u, rows_v, ob0, ob1, w0_v, w1_v, bp_v):
    _, _, wid = _worker_id()
    ch = idx_s.shape[0]
    d2 = u_t.shape[1]
    epw = ps.shape[0] // NW
    nch = epw // ch

    pltpu.sync_copy(w0, w0_v)
    pltpu.sync_copy(w1, w1_v)
    pltpu.sync_copy(bp, bp_v)

    iota = lax.iota(jnp.int32, L)
    zv = jnp.zeros((L,), F32)
    base0 = wid * epw

    def chunk(c, _):
        b = base0 + c * ch
        pltpu.sync_copy(ps.at[pl.ds(b, ch)], idx_s)
        pltpu.sync_copy(pd.at[pl.ds(b, ch)], idx_d)
        pltpu.sync_copy(u_t.at[idx_s], rows_u)
        pltpu.sync_copy(v_t.at[idx_d], rows_v)

        def group(g, _):
            eids = iota + g * L

            def feat(fq, accs):
                a0, a1 = accs
                for u in range(4):
                    f = fq * 4 + u
                    f16 = jnp.full((L,), f, jnp.int32)
                    uv = plsc.load_gather(rows_u, [eids, f16])
                    vv = plsc.load_gather(rows_v, [eids, f16])
                    t = jnp.maximum(uv + vv, 0.0)
                    a0 = a0 + t * w0_v[f, pl.ds(0, L)]
                    a1 = a1 + t * w1_v[f, pl.ds(0, L)]
                return (a0, a1)

            a0, a1 = lax.fori_loop(0, d2 // 4, feat, (zv, zv))
            ob0[pl.ds(g * L, L)] = a0 + bp_v[0, pl.ds(0, L)]
            ob1[pl.ds(g * L, L)] = a1 + bp_v[1, pl.ds(0, L)]
            return 0

        lax.fori_loop(0, ch // L, group, 0)
        pltpu.sync_copy(ob0, o0.at[pl.ds(b, ch)])
        pltpu.sync_copy(ob1, o1.at[pl.ds(b, ch)])
        return 0

    lax.fori_loop(0, nch, chunk, 0)


def _tc1_body(x, aggp, degp, w1s, w1a, b1, we1a, we1b, be1,
              h1_o, a_o, b_o):
    agg = aggp[0] + aggp[1]
    deg = jnp.sum(degp[...], axis=0)
    aggn = agg / jnp.maximum(deg, 1.0)[:, None]
    h1 = jnp.tanh(jnp.dot(x[...], w1s[...], preferred_element_type=F32)
                  + jnp.dot(aggn, w1a[...], preferred_element_type=F32)
                  + b1[...])
    h1_o[...] = h1
    a_o[...] = jnp.dot(h1, we1a[...], preferred_element_type=F32)
    b_o[...] = jnp.dot(h1, we1b[...], preferred_element_type=F32) + be1[...]


def _tc2_body(h1, aggp, degp, w2s, w2a, b2, wp1a, wp1b, wp1c, wp1d, bp1,
              u_o, v_o):
    agg = aggp[0] + aggp[1]
    deg = jnp.sum(degp[...], axis=0)
    aggn = agg / jnp.maximum(deg, 1.0)[:, None]
    h1v = h1[...]
    h2 = jnp.tanh(jnp.dot(h1v, w2s[...], preferred_element_type=F32)
                  + jnp.dot(aggn, w2a[...], preferred_element_type=F32)
                  + b2[...])
    u_o[...] = (jnp.dot(h1v, wp1a[...], preferred_element_type=F32)
                + jnp.dot(h2, wp1b[...], preferred_element_type=F32)
                + bp1[...])
    v_o[...] = (jnp.dot(h1v, wp1c[...], preferred_element_type=F32)
                + jnp.dot(h2, wp1d[...], preferred_element_type=F32))


def kernel(x, edge_index, edge_weight, pred_edge_index,
           W1s, W1a, b1, We1, be1, We2, be2,
           W2s, W2a, b2, Wp1, bp1, Wp2, bp2):
    n, d_in = x.shape
    h = W1s.shape[1]
    e = edge_index.shape[1]
    p = pred_edge_index.shape[1]
    d2 = 2 * h
    src, dst = edge_index[0], edge_index[1]
    pps, ppd = pred_edge_index[0], pred_edge_index[1]

    npad = ((n + 511) // 512) * 512
    while (npad // NS) % CH:
        npad += 512

    sc_agg = pl.kernel(
        _sc_agg_body,
        out_type=[jax.ShapeDtypeStruct((NC, npad, h), F32),
                  jax.ShapeDtypeStruct((NW, npad), F32)],
        mesh=_sc_mesh(),
        compiler_params=_SC_PARAMS,
        scratch_types=[
            pltpu.VMEM((CH,), jnp.int32), pltpu.VMEM((CH,), jnp.int32),
            pltpu.VMEM((CH,), F32), pltpu.VMEM((CH, h), F32),
            pltpu.VMEM((npad,), F32),
            pltpu.VMEM_SHARED((npad, h), F32),
        ],
    )
    aggp1, degp1 = sc_agg(x, src, dst, edge_weight)

    rb = 512
    gn = pl.cdiv(n, rb)
    wspec = pl.BlockSpec((d_in, h), lambda i: (0, 0))
    bspec = pl.BlockSpec((1, h), lambda i: (0, 0))
    nspec = pl.BlockSpec((rb, h), lambda i: (i, 0))
    aggspec = pl.BlockSpec((NC, rb, h), lambda i: (0, i, 0))
    degspec = pl.BlockSpec((NW, rb), lambda i: (0, i))

    h1, a_t, b_t = pl.pallas_call(
        _tc1_body,
        grid=(gn,),
        in_specs=[nspec, aggspec, degspec, wspec, wspec, bspec,
                  wspec, wspec, bspec],
        out_specs=[nspec, nspec, nspec],
        out_shape=[jax.ShapeDtypeStruct((n, h), F32)] * 3,
    )(x, aggp1, degp1, W1s, W1a, b1.reshape(1, h),
      We1[:h], We1[h:], be1.reshape(1, h))

    we2b = jnp.broadcast_to(We2.reshape(h, 1), (h, L))
    be2b = jnp.broadcast_to(be2.reshape(1), (L,))
    sc_rw = pl.kernel(
        _sc_rw_body,
        out_type=[jax.ShapeDtypeStruct((e,), F32),
                  jax.ShapeDtypeStruct((NW, L), F32)],
        mesh=_sc_mesh(),
        compiler_params=_SC_PARAMS,
        scratch_types=[
            pltpu.VMEM((CH,), jnp.int32), pltpu.VMEM((CH,), jnp.int32),
            pltpu.VMEM((CH,), F32),
            pltpu.VMEM((CH,), jnp.int32), pltpu.VMEM((CH,), jnp.int32),
            pltpu.VMEM((CH,), F32),
            pltpu.VMEM((CH, h), F32), pltpu.VMEM((CH, h), F32),
            pltpu.VMEM((CH, h), F32), pltpu.VMEM((CH, h), F32),
            pltpu.VMEM((CH,), F32), pltpu.VMEM((h, L), F32),
            pltpu.VMEM((L,), F32), pltpu.VMEM((L,), F32),
            pltpu.SemaphoreType.DMA, pltpu.SemaphoreType.DMA,
            pltpu.SemaphoreType.DMA,
        ],
    )
    rw, l1p = sc_rw(a_t, b_t, src, dst, edge_weight, we2b, be2b)

    sc_agg2 = pl.kernel(
        _sc_agg_body,
        out_type=[jax.ShapeDtypeStruct((NC, npad, h), F32),
                  jax.ShapeDtypeStruct((NW, npad), F32)],
        mesh=_sc_mesh(),
        compiler_params=_SC_PARAMS,
        scratch_types=[
            pltpu.VMEM((CH,), jnp.int32), pltpu.VMEM((CH,), jnp.int32),
            pltpu.VMEM((CH,), F32), pltpu.VMEM((CH, h), F32),
            pltpu.VMEM((npad,), F32),
            pltpu.VMEM_SHARED((npad, h), F32),
        ],
    )
    aggp2, degp2 = sc_agg2(h1, src, dst, rw)

    n2spec = pl.BlockSpec((rb, d2), lambda i: (i, 0))
    w2spec = pl.BlockSpec((h, d2), lambda i: (0, 0))
    b2spec = pl.BlockSpec((1, d2), lambda i: (0, 0))
    u_t, v_t = pl.pallas_call(
        _tc2_body,
        grid=(gn,),
        in_specs=[nspec, aggspec, degspec, wspec, wspec, bspec,
                  w2spec, w2spec, w2spec, w2spec, b2spec],
        out_specs=[n2spec, n2spec],
        out_shape=[jax.ShapeDtypeStruct((n, d2), F32)] * 2,
    )(h1, aggp2, degp2, W2s, W2a, b2.reshape(1, h),
      Wp1[:h], Wp1[h:d2], Wp1[d2:d2 + h], Wp1[d2 + h:], bp1.reshape(1, d2))

    w0b = jnp.broadcast_to(Wp2[:, 0:1], (d2, L))
    w1b = jnp.broadcast_to(Wp2[:, 1:2], (d2, L))
    bpb = jnp.broadcast_to(bp2.reshape(2, 1), (2, L))
    sc_pred = pl.kernel(
        _sc_pred_body,
        out_type=[jax.ShapeDtypeStruct((p,), F32),
                  jax.ShapeDtypeStruct((p,), F32)],
        mesh=_sc_mesh(),
        compiler_params=_SC_PARAMS,
        scratch_types=[
            pltpu.VMEM((CH,), jnp.int32), pltpu.VMEM((CH,), jnp.int32),
            pltpu.VMEM((CH, d2), F32), pltpu.VMEM((CH, d2), F32),
            pltpu.VMEM((CH,), F32), pltpu.VMEM((CH,), F32),
            pltpu.VMEM((d2, L), F32), pltpu.VMEM((d2, L), F32),
            pltpu.VMEM((2, L), F32),
        ],
    )
    o0, o1 = sc_pred(u_t, v_t, pps, ppd, w0b, w1b, bpb)

    logits = jnp.stack([o0, o1], axis=1)
    l1_reg = jnp.sum(l1p) / e
    return logits, l1_reg
